# Initial kernel scaffold; baseline (speedup 1.0000x reference)
import jax
import jax.numpy as jnp
from jax.experimental import pallas as pl


def kernel(x, k):
    raise NotImplementedError("write your pallas kernel here")


# speedup vs baseline: 1.2104x; 1.2104x over previous
import jax
import jax.numpy as jnp
from jax import lax
from jax.experimental import pallas as pl
from jax.experimental.pallas import tpu as pltpu

_K = 1024
_W = 32
_TT = 512


def _vq_tile_kernel(x_ref, k_ref, xl_ref, xd_ref, sx_ref, sx2_ref, smd_ref,
                    scm_ref):
    xb = x_ref[0]
    kf = k_ref[...]

    rn = jnp.sum(xb * xb, axis=0, keepdims=True)
    kn = jnp.sum(kf * kf, axis=1, keepdims=True)
    scores = lax.dot_general(kf, xb, (((1,), (0,)), ((), ())))
    d = (rn - 2.0 * scores) + kn

    min_d = jnp.min(d, axis=0, keepdims=True)
    iota = lax.broadcasted_iota(jnp.int32, (_K, _TT), 0)
    idx = jnp.min(jnp.where(d == min_d, iota, _K), axis=0, keepdims=True)
    xl_ref[0] = idx

    onehot = (iota == idx).astype(jnp.float32)
    g = lax.dot_general(kf, onehot, (((0,), (0,)), ((), ())),
                        precision=lax.Precision.HIGHEST)
    diff = g - xb
    xd_ref[0] = xb + diff

    first = (pl.program_id(0) == 0) & (pl.program_id(1) == 0)

    @pl.when(first)
    def _init():
        sx_ref[0, 0] = 0.0
        sx2_ref[0, 0] = 0.0
        smd_ref[0, 0] = 0.0
        scm_ref[0, 0] = 0.0

    sx_ref[0, 0] += jnp.sum(xb)
    sx2_ref[0, 0] += jnp.sum(rn)
    smd_ref[0, 0] += jnp.sum(min_d)
    scm_ref[0, 0] += jnp.sum(diff * diff)


def kernel(x, k):
    n, width, t = x.shape
    n_t_tiles = t // _TT
    grid = (n, n_t_tiles)

    out_types = (
        jax.ShapeDtypeStruct((n * n_t_tiles, 1, _TT), jnp.int32),
        jax.ShapeDtypeStruct((n, width, t), jnp.float32),
        jax.ShapeDtypeStruct((1, 1), jnp.float32),
        jax.ShapeDtypeStruct((1, 1), jnp.float32),
        jax.ShapeDtypeStruct((1, 1), jnp.float32),
        jax.ShapeDtypeStruct((1, 1), jnp.float32),
    )
    smem_spec = pl.BlockSpec(memory_space=pltpu.SMEM)
    xl_tiles, x_d, sx, sx2, smd, scm = pl.pallas_call(
        _vq_tile_kernel,
        grid=grid,
        in_specs=[
            pl.BlockSpec((1, width, _TT), lambda i, j: (i, 0, j)),
            pl.BlockSpec((_K, width), lambda i, j: (0, 0)),
        ],
        out_specs=(
            pl.BlockSpec((1, 1, _TT), lambda i, j, nt=n_t_tiles: (i * nt + j, 0, 0)),
            pl.BlockSpec((1, width, _TT), lambda i, j: (i, 0, j)),
            smem_spec, smem_spec, smem_spec, smem_spec,
        ),
        out_shape=out_types,
    )(x, k)

    x_l = xl_tiles.reshape(n, t)
    total = jnp.float32(n * t * width)
    n_rows = jnp.float32(n * t)
    mean = sx[0, 0] / total
    prenorm = jnp.sqrt(jnp.maximum(sx2[0, 0] / total - mean * mean, 0.0))
    fit = smd[0, 0] / n_rows
    commit_loss = scm[0, 0] / total
    return (x_l, x_d, commit_loss, fit, prenorm)


# speedup vs baseline: 1.3452x; 1.1114x over previous
import jax
import jax.numpy as jnp
from jax import lax
from jax.experimental import pallas as pl
from jax.experimental.pallas import tpu as pltpu

_K = 1024
_W = 32
_TT = 512


def _vq_tile_kernel(x_ref, k_ref, xl_ref, xd_ref, sx_ref, sx2_ref, smd_ref,
                    scm_ref):
    xb = x_ref[0]
    kf = k_ref[...]

    kn = jnp.sum(kf * kf, axis=1, keepdims=True)
    rn = jnp.sum(xb * xb, axis=0, keepdims=True)
    scores = lax.dot_general(kf, xb, (((1,), (0,)), ((), ())))
    d = (rn - 2.0 * scores) + kn

    min_d = jnp.min(d, axis=0, keepdims=True)
    onehot = (d == min_d).astype(jnp.float32)

    iota_col = lax.broadcasted_iota(jnp.int32, (_K, 1), 0).astype(jnp.float32)
    ones_col = jnp.ones((_K, 1), jnp.float32)
    k_gather = jnp.concatenate(
        [kf, iota_col, iota_col * iota_col, ones_col], axis=1)
    g_aug = lax.dot_general(k_gather, onehot, (((0,), (0,)), ((), ())),
                            precision=lax.Precision.HIGHEST)
    g = g_aug[:_W, :]
    s = g_aug[_W:_W + 1, :]
    q = g_aug[_W + 1:_W + 2, :]
    c = g_aug[_W + 2:_W + 3, :]
    delta = jnp.sqrt(jnp.maximum(2.0 * q - s * s, 0.0))
    idx_f = jnp.where(c > 1.5, 0.5 * (s - delta), s)
    idx = idx_f.astype(jnp.int32)

    xl_ref[0] = idx
    g = g / c
    diff = g - xb
    xd_ref[0] = xb + diff

    first = (pl.program_id(0) == 0) & (pl.program_id(1) == 0)

    @pl.when(first)
    def _init():
        sx_ref[0, 0] = 0.0
        sx2_ref[0, 0] = 0.0
        smd_ref[0, 0] = 0.0
        scm_ref[0, 0] = 0.0

    sx_ref[0, 0] += jnp.sum(xb)
    sx2_ref[0, 0] += jnp.sum(rn)
    smd_ref[0, 0] += jnp.sum(min_d)
    scm_ref[0, 0] += jnp.sum(diff * diff)


def kernel(x, k):
    n, width, t = x.shape
    n_t_tiles = t // _TT
    grid = (n, n_t_tiles)

    out_types = (
        jax.ShapeDtypeStruct((n * n_t_tiles, 1, _TT), jnp.int32),
        jax.ShapeDtypeStruct((n, width, t), jnp.float32),
        jax.ShapeDtypeStruct((1, 1), jnp.float32),
        jax.ShapeDtypeStruct((1, 1), jnp.float32),
        jax.ShapeDtypeStruct((1, 1), jnp.float32),
        jax.ShapeDtypeStruct((1, 1), jnp.float32),
    )
    smem_spec = pl.BlockSpec(memory_space=pltpu.SMEM)
    xl_tiles, x_d, sx, sx2, smd, scm = pl.pallas_call(
        _vq_tile_kernel,
        grid=grid,
        in_specs=[
            pl.BlockSpec((1, width, _TT), lambda i, j: (i, 0, j)),
            pl.BlockSpec((_K, width), lambda i, j: (0, 0)),
        ],
        out_specs=(
            pl.BlockSpec((1, 1, _TT), lambda i, j, nt=n_t_tiles: (i * nt + j, 0, 0)),
            pl.BlockSpec((1, width, _TT), lambda i, j: (i, 0, j)),
            smem_spec, smem_spec, smem_spec, smem_spec,
        ),
        out_shape=out_types,
    )(x, k)

    x_l = xl_tiles.reshape(n, t)
    total = jnp.float32(n * t * width)
    n_rows = jnp.float32(n * t)
    mean = sx[0, 0] / total
    prenorm = jnp.sqrt(jnp.maximum(sx2[0, 0] / total - mean * mean, 0.0))
    fit = smd[0, 0] / n_rows
    commit_loss = scm[0, 0] / total
    return (x_l, x_d, commit_loss, fit, prenorm)


# speedup vs baseline: 2.2978x; 1.7081x over previous
import jax
import jax.numpy as jnp
from jax import lax
from jax.experimental import pallas as pl
from jax.experimental.pallas import tpu as pltpu

_K = 1024
_W = 32
_TT = 512


def _vq_tile_kernel(x_ref, k_ref, xl_ref, xd_ref, sx_ref, sx2_ref, smd_ref,
                    scm_ref):
    xb = x_ref[0]
    kf = k_ref[...]

    kn = jnp.sum(kf * kf, axis=1, keepdims=True)
    rn = jnp.sum(xb * xb, axis=0, keepdims=True)
    scores = lax.dot_general(kf, xb, (((1,), (0,)), ((), ())))
    d = (rn - 2.0 * scores) + kn

    min_d = jnp.min(d, axis=0, keepdims=True)
    onehot = (d == min_d).astype(jnp.bfloat16)

    iota = lax.broadcasted_iota(jnp.int32, (_K, 1), 0)
    j_hi = (iota & ~31).astype(jnp.float32)
    j_lo = (iota & 31).astype(jnp.float32)
    jsq = iota * iota
    q_hi = (jsq & (127 << 14)).astype(jnp.float32)
    q_mid = (jsq & (127 << 7)).astype(jnp.float32)
    q_lo = (jsq & 127).astype(jnp.float32)
    ones_col = jnp.ones((_K, 1), jnp.float32)
    k_gather = jnp.concatenate(
        [kf, j_hi, j_lo, q_hi, q_mid, q_lo, ones_col],
        axis=1).astype(jnp.bfloat16)
    g_aug = lax.dot_general(k_gather, onehot, (((0,), (0,)), ((), ())),
                            preferred_element_type=jnp.float32)
    g = g_aug[:_W, :]
    s = g_aug[_W:_W + 1, :] + g_aug[_W + 1:_W + 2, :]
    q = (g_aug[_W + 2:_W + 3, :] + g_aug[_W + 3:_W + 4, :]) + g_aug[_W + 4:_W + 5, :]
    c = g_aug[_W + 5:_W + 6, :]
    delta = jnp.sqrt(jnp.maximum(2.0 * q - s * s, 0.0))
    idx_f = jnp.where(c > 1.5, 0.5 * (s - delta), s)
    idx = idx_f.astype(jnp.int32)

    xl_ref[0] = idx
    g = g / c
    diff = g - xb
    xd_ref[0] = xb + diff

    first = (pl.program_id(0) == 0) & (pl.program_id(1) == 0)

    @pl.when(first)
    def _init():
        sx_ref[0, 0] = 0.0
        sx2_ref[0, 0] = 0.0
        smd_ref[0, 0] = 0.0
        scm_ref[0, 0] = 0.0

    sx_ref[0, 0] += jnp.sum(xb)
    sx2_ref[0, 0] += jnp.sum(rn)
    smd_ref[0, 0] += jnp.sum(min_d)
    scm_ref[0, 0] += jnp.sum(diff * diff)


def kernel(x, k):
    n, width, t = x.shape
    n_t_tiles = t // _TT
    grid = (n, n_t_tiles)

    out_types = (
        jax.ShapeDtypeStruct((n * n_t_tiles, 1, _TT), jnp.int32),
        jax.ShapeDtypeStruct((n, width, t), jnp.float32),
        jax.ShapeDtypeStruct((1, 1), jnp.float32),
        jax.ShapeDtypeStruct((1, 1), jnp.float32),
        jax.ShapeDtypeStruct((1, 1), jnp.float32),
        jax.ShapeDtypeStruct((1, 1), jnp.float32),
    )
    smem_spec = pl.BlockSpec(memory_space=pltpu.SMEM)
    xl_tiles, x_d, sx, sx2, smd, scm = pl.pallas_call(
        _vq_tile_kernel,
        grid=grid,
        in_specs=[
            pl.BlockSpec((1, width, _TT), lambda i, j: (i, 0, j)),
            pl.BlockSpec((_K, width), lambda i, j: (0, 0)),
        ],
        out_specs=(
            pl.BlockSpec((1, 1, _TT), lambda i, j, nt=n_t_tiles: (i * nt + j, 0, 0)),
            pl.BlockSpec((1, width, _TT), lambda i, j: (i, 0, j)),
            smem_spec, smem_spec, smem_spec, smem_spec,
        ),
        out_shape=out_types,
    )(x, k)

    x_l = xl_tiles.reshape(n, t)
    total = jnp.float32(n * t * width)
    n_rows = jnp.float32(n * t)
    mean = sx[0, 0] / total
    prenorm = jnp.sqrt(jnp.maximum(sx2[0, 0] / total - mean * mean, 0.0))
    fit = smd[0, 0] / n_rows
    commit_loss = scm[0, 0] / total
    return (x_l, x_d, commit_loss, fit, prenorm)


# speedup vs baseline: 3.1532x; 1.3722x over previous
import jax
import jax.numpy as jnp
from jax import lax
from jax.experimental import pallas as pl
from jax.experimental.pallas import tpu as pltpu

_K = 1024
_W = 32
_TT = 1024


def _vq_tile_kernel(x_ref, k_ref, xl_ref, xd_ref, sx_ref, sx2_ref, smd_ref,
                    scm_ref):
    xb = x_ref[0]
    kf = k_ref[...]

    kn = jnp.sum(kf * kf, axis=1, keepdims=True)
    rn = jnp.sum(xb * xb, axis=0, keepdims=True)
    scores = lax.dot_general(kf, xb, (((1,), (0,)), ((), ())))
    d = (rn - 2.0 * scores) + kn

    min_d = jnp.min(d, axis=0, keepdims=True)
    onehot = (d == min_d).astype(jnp.bfloat16)

    iota = lax.broadcasted_iota(jnp.int32, (_K, 1), 0)
    j_hi = (iota & ~31).astype(jnp.float32)
    j_lo = (iota & 31).astype(jnp.float32)
    jsq = iota * iota
    q_hi = (jsq & (127 << 14)).astype(jnp.float32)
    q_mid = (jsq & (127 << 7)).astype(jnp.float32)
    q_lo = (jsq & 127).astype(jnp.float32)
    ones_col = jnp.ones((_K, 1), jnp.float32)
    k_gather = jnp.concatenate(
        [kf, j_hi, j_lo, q_hi, q_mid, q_lo, ones_col],
        axis=1).astype(jnp.bfloat16)
    g_aug = lax.dot_general(k_gather, onehot, (((0,), (0,)), ((), ())),
                            preferred_element_type=jnp.float32)
    g = g_aug[:_W, :]
    s = g_aug[_W:_W + 1, :] + g_aug[_W + 1:_W + 2, :]
    q = (g_aug[_W + 2:_W + 3, :] + g_aug[_W + 3:_W + 4, :]) + g_aug[_W + 4:_W + 5, :]
    c = g_aug[_W + 5:_W + 6, :]
    delta = jnp.sqrt(jnp.maximum(2.0 * q - s * s, 0.0))
    idx_f = jnp.where(c > 1.5, 0.5 * (s - delta), s)
    idx = idx_f.astype(jnp.int32)

    xl_ref[0] = idx
    g = g / c
    diff = g - xb
    xd_ref[0] = xb + diff

    first = (pl.program_id(0) == 0) & (pl.program_id(1) == 0)

    @pl.when(first)
    def _init():
        sx_ref[0, 0] = 0.0
        sx2_ref[0, 0] = 0.0
        smd_ref[0, 0] = 0.0
        scm_ref[0, 0] = 0.0

    sx_ref[0, 0] += jnp.sum(xb)
    sx2_ref[0, 0] += jnp.sum(rn)
    smd_ref[0, 0] += jnp.sum(min_d)
    scm_ref[0, 0] += jnp.sum(diff * diff)


def kernel(x, k):
    n, width, t = x.shape
    n_t_tiles = t // _TT
    grid = (n, n_t_tiles)

    out_types = (
        jax.ShapeDtypeStruct((n * n_t_tiles, 1, _TT), jnp.int32),
        jax.ShapeDtypeStruct((n, width, t), jnp.float32),
        jax.ShapeDtypeStruct((1, 1), jnp.float32),
        jax.ShapeDtypeStruct((1, 1), jnp.float32),
        jax.ShapeDtypeStruct((1, 1), jnp.float32),
        jax.ShapeDtypeStruct((1, 1), jnp.float32),
    )
    smem_spec = pl.BlockSpec(memory_space=pltpu.SMEM)
    xl_tiles, x_d, sx, sx2, smd, scm = pl.pallas_call(
        _vq_tile_kernel,
        grid=grid,
        in_specs=[
            pl.BlockSpec((1, width, _TT), lambda i, j: (i, 0, j)),
            pl.BlockSpec((_K, width), lambda i, j: (0, 0)),
        ],
        out_specs=(
            pl.BlockSpec((1, 1, _TT), lambda i, j, nt=n_t_tiles: (i * nt + j, 0, 0)),
            pl.BlockSpec((1, width, _TT), lambda i, j: (i, 0, j)),
            smem_spec, smem_spec, smem_spec, smem_spec,
        ),
        out_shape=out_types,
    )(x, k)

    x_l = xl_tiles.reshape(n, t)
    total = jnp.float32(n * t * width)
    n_rows = jnp.float32(n * t)
    mean = sx[0, 0] / total
    prenorm = jnp.sqrt(jnp.maximum(sx2[0, 0] / total - mean * mean, 0.0))
    fit = smd[0, 0] / n_rows
    commit_loss = scm[0, 0] / total
    return (x_l, x_d, commit_loss, fit, prenorm)


# speedup vs baseline: 3.7757x; 1.1974x over previous
import jax
import jax.numpy as jnp
from jax import lax
from jax.experimental import pallas as pl
from jax.experimental.pallas import tpu as pltpu

_K = 1024
_W = 32
_TT = 2048


def _vq_tile_kernel(x_ref, k_ref, xl_ref, xd_ref, sx_ref, sx2_ref, smd_ref,
                    scm_ref):
    xb = x_ref[0]
    kf = k_ref[...]

    kn = jnp.sum(kf * kf, axis=1, keepdims=True)
    rn = jnp.sum(xb * xb, axis=0, keepdims=True)
    scores = lax.dot_general(kf, xb, (((1,), (0,)), ((), ())))
    d = (rn - 2.0 * scores) + kn

    min_d = jnp.min(d, axis=0, keepdims=True)
    onehot = (d == min_d).astype(jnp.bfloat16)

    iota = lax.broadcasted_iota(jnp.int32, (_K, 1), 0)
    j_hi = (iota & ~31).astype(jnp.float32)
    j_lo = (iota & 31).astype(jnp.float32)
    jsq = iota * iota
    q_hi = (jsq & (127 << 14)).astype(jnp.float32)
    q_mid = (jsq & (127 << 7)).astype(jnp.float32)
    q_lo = (jsq & 127).astype(jnp.float32)
    ones_col = jnp.ones((_K, 1), jnp.float32)
    k_gather = jnp.concatenate(
        [kf, j_hi, j_lo, q_hi, q_mid, q_lo, ones_col],
        axis=1).astype(jnp.bfloat16)
    g_aug = lax.dot_general(k_gather, onehot, (((0,), (0,)), ((), ())),
                            preferred_element_type=jnp.float32)
    g = g_aug[:_W, :]
    s = g_aug[_W:_W + 1, :] + g_aug[_W + 1:_W + 2, :]
    q = (g_aug[_W + 2:_W + 3, :] + g_aug[_W + 3:_W + 4, :]) + g_aug[_W + 4:_W + 5, :]
    c = g_aug[_W + 5:_W + 6, :]
    delta = jnp.sqrt(jnp.maximum(2.0 * q - s * s, 0.0))
    idx_f = jnp.where(c > 1.5, 0.5 * (s - delta), s)
    idx = idx_f.astype(jnp.int32)

    xl_ref[0] = idx
    g = g / c
    diff = g - xb
    xd_ref[0] = xb + diff

    first = (pl.program_id(0) == 0) & (pl.program_id(1) == 0)

    @pl.when(first)
    def _init():
        sx_ref[0, 0] = 0.0
        sx2_ref[0, 0] = 0.0
        smd_ref[0, 0] = 0.0
        scm_ref[0, 0] = 0.0

    sx_ref[0, 0] += jnp.sum(xb)
    sx2_ref[0, 0] += jnp.sum(rn)
    smd_ref[0, 0] += jnp.sum(min_d)
    scm_ref[0, 0] += jnp.sum(diff * diff)


def kernel(x, k):
    n, width, t = x.shape
    n_t_tiles = t // _TT
    grid = (n, n_t_tiles)

    out_types = (
        jax.ShapeDtypeStruct((n * n_t_tiles, 1, _TT), jnp.int32),
        jax.ShapeDtypeStruct((n, width, t), jnp.float32),
        jax.ShapeDtypeStruct((1, 1), jnp.float32),
        jax.ShapeDtypeStruct((1, 1), jnp.float32),
        jax.ShapeDtypeStruct((1, 1), jnp.float32),
        jax.ShapeDtypeStruct((1, 1), jnp.float32),
    )
    smem_spec = pl.BlockSpec(memory_space=pltpu.SMEM)
    xl_tiles, x_d, sx, sx2, smd, scm = pl.pallas_call(
        _vq_tile_kernel,
        grid=grid,
        in_specs=[
            pl.BlockSpec((1, width, _TT), lambda i, j: (i, 0, j)),
            pl.BlockSpec((_K, width), lambda i, j: (0, 0)),
        ],
        out_specs=(
            pl.BlockSpec((1, 1, _TT), lambda i, j, nt=n_t_tiles: (i * nt + j, 0, 0)),
            pl.BlockSpec((1, width, _TT), lambda i, j: (i, 0, j)),
            smem_spec, smem_spec, smem_spec, smem_spec,
        ),
        out_shape=out_types,
    )(x, k)

    x_l = xl_tiles.reshape(n, t)
    total = jnp.float32(n * t * width)
    n_rows = jnp.float32(n * t)
    mean = sx[0, 0] / total
    prenorm = jnp.sqrt(jnp.maximum(sx2[0, 0] / total - mean * mean, 0.0))
    fit = smd[0, 0] / n_rows
    commit_loss = scm[0, 0] / total
    return (x_l, x_d, commit_loss, fit, prenorm)


# speedup vs baseline: 4.1013x; 1.0862x over previous
import jax
import jax.numpy as jnp
from jax import lax
from jax.experimental import pallas as pl
from jax.experimental.pallas import tpu as pltpu

_K = 1024
_W = 32
_TT = 4096


def _vq_tile_kernel(x_ref, k_ref, xl_ref, xd_ref, sx_ref, sx2_ref, smd_ref,
                    scm_ref):
    xb = x_ref[0]
    kf = k_ref[...]

    kn = jnp.sum(kf * kf, axis=1, keepdims=True)
    rn = jnp.sum(xb * xb, axis=0, keepdims=True)
    scores = lax.dot_general(kf, xb, (((1,), (0,)), ((), ())))
    d = (rn - 2.0 * scores) + kn

    min_d = jnp.min(d, axis=0, keepdims=True)
    onehot = (d == min_d).astype(jnp.bfloat16)

    iota = lax.broadcasted_iota(jnp.int32, (_K, 1), 0)
    j_hi = (iota & ~31).astype(jnp.float32)
    j_lo = (iota & 31).astype(jnp.float32)
    jsq = iota * iota
    q_hi = (jsq & (127 << 14)).astype(jnp.float32)
    q_mid = (jsq & (127 << 7)).astype(jnp.float32)
    q_lo = (jsq & 127).astype(jnp.float32)
    ones_col = jnp.ones((_K, 1), jnp.float32)
    k_gather = jnp.concatenate(
        [kf, j_hi, j_lo, q_hi, q_mid, q_lo, ones_col],
        axis=1).astype(jnp.bfloat16)
    g_aug = lax.dot_general(k_gather, onehot, (((0,), (0,)), ((), ())),
                            preferred_element_type=jnp.float32)
    g = g_aug[:_W, :]
    s = g_aug[_W:_W + 1, :] + g_aug[_W + 1:_W + 2, :]
    q = (g_aug[_W + 2:_W + 3, :] + g_aug[_W + 3:_W + 4, :]) + g_aug[_W + 4:_W + 5, :]
    c = g_aug[_W + 5:_W + 6, :]
    delta = jnp.sqrt(jnp.maximum(2.0 * q - s * s, 0.0))
    idx_f = jnp.where(c > 1.5, 0.5 * (s - delta), s)
    idx = idx_f.astype(jnp.int32)

    xl_ref[0] = idx
    g = g / c
    diff = g - xb
    xd_ref[0] = xb + diff

    first = (pl.program_id(0) == 0) & (pl.program_id(1) == 0)

    @pl.when(first)
    def _init():
        sx_ref[0, 0] = 0.0
        sx2_ref[0, 0] = 0.0
        smd_ref[0, 0] = 0.0
        scm_ref[0, 0] = 0.0

    sx_ref[0, 0] += jnp.sum(xb)
    sx2_ref[0, 0] += jnp.sum(rn)
    smd_ref[0, 0] += jnp.sum(min_d)
    scm_ref[0, 0] += jnp.sum(diff * diff)


def kernel(x, k):
    n, width, t = x.shape
    n_t_tiles = t // _TT
    grid = (n, n_t_tiles)

    out_types = (
        jax.ShapeDtypeStruct((n * n_t_tiles, 1, _TT), jnp.int32),
        jax.ShapeDtypeStruct((n, width, t), jnp.float32),
        jax.ShapeDtypeStruct((1, 1), jnp.float32),
        jax.ShapeDtypeStruct((1, 1), jnp.float32),
        jax.ShapeDtypeStruct((1, 1), jnp.float32),
        jax.ShapeDtypeStruct((1, 1), jnp.float32),
    )
    smem_spec = pl.BlockSpec(memory_space=pltpu.SMEM)
    xl_tiles, x_d, sx, sx2, smd, scm = pl.pallas_call(
        _vq_tile_kernel,
        grid=grid,
        in_specs=[
            pl.BlockSpec((1, width, _TT), lambda i, j: (i, 0, j)),
            pl.BlockSpec((_K, width), lambda i, j: (0, 0)),
        ],
        out_specs=(
            pl.BlockSpec((1, 1, _TT), lambda i, j, nt=n_t_tiles: (i * nt + j, 0, 0)),
            pl.BlockSpec((1, width, _TT), lambda i, j: (i, 0, j)),
            smem_spec, smem_spec, smem_spec, smem_spec,
        ),
        out_shape=out_types,
    )(x, k)

    x_l = xl_tiles.reshape(n, t)
    total = jnp.float32(n * t * width)
    n_rows = jnp.float32(n * t)
    mean = sx[0, 0] / total
    prenorm = jnp.sqrt(jnp.maximum(sx2[0, 0] / total - mean * mean, 0.0))
    fit = smd[0, 0] / n_rows
    commit_loss = scm[0, 0] / total
    return (x_l, x_d, commit_loss, fit, prenorm)
